# baseline (speedup 1.0000x reference)
import functools

import jax
import jax.numpy as jnp
from jax import lax
from jax.experimental import pallas as pl
from jax.experimental.pallas import tpu as pltpu
from jax.experimental.pallas import tpu_sc as plsc

_NC, _NS = 2, 16
_NW = _NC * _NS


def kernel(inputs, pos_embedding):
    B, seq_len = inputs.shape
    D = pos_embedding.shape[1]
    table = pos_embedding[:seq_len]
    rows_w = seq_len // _NW
    CH = 32
    nch = rows_w // CH
    mesh = plsc.VectorSubcoreMesh(
        core_axis_name="c", subcore_axis_name="s",
        num_cores=_NC, num_subcores=_NS)

    @functools.partial(
        pl.kernel,
        out_type=jax.ShapeDtypeStruct((B, seq_len, D), jnp.float32),
        mesh=mesh,
        scratch_types=[
            pltpu.VMEM((2, CH, D), jnp.float32),
            pltpu.SemaphoreType.DMA,
            pltpu.SemaphoreType.DMA,
        ],
    )
    def sc_copy(table_hbm, out_hbm, buf, in_sem, out_sem):
        wid = lax.axis_index("s") * _NC + lax.axis_index("c")
        base = wid * rows_w

        def gather(c):
            return pltpu.async_copy(
                table_hbm.at[pl.ds(base + c * CH, CH), :],
                buf.at[c % 2], in_sem)

        gathers = {0: gather(0)}
        prev_outs = []
        for c in range(nch):
            for o in prev_outs:
                o.wait()
            if c + 1 < nch:
                gathers[c + 1] = gather(c + 1)
            gathers[c].wait()
            prev_outs = [
                pltpu.async_copy(
                    buf.at[c % 2],
                    out_hbm.at[b, pl.ds(base + c * CH, CH), :],
                    out_sem)
                for b in range(B)
            ]
        for o in prev_outs:
            o.wait()

    return sc_copy(table)
